# baseline (speedup 1.0000x reference)
import functools
import jax
import jax.numpy as jnp
from jax import lax
from jax.experimental import pallas as pl
from jax.experimental.pallas import tpu as pltpu
from jax.experimental.pallas import tpu_sc as plsc

B, S, D = 4, 8192, 768
N = B * S
NC, NS, LANES = 2, 16, 16
NW = NC * NS
ROWS_PER_TILE = N // NW
CHUNK = 64


def _sc_body(x_hbm, idx_hbm, w_hbm, o_hbm, idx_v, gat_v, x_v, gsem, xsem):
    wid = lax.axis_index("s") * NC + lax.axis_index("c")
    base = wid * ROWS_PER_TILE
    pltpu.sync_copy(idx_hbm.at[pl.ds(base, ROWS_PER_TILE)], idx_v)

    @pl.loop(0, ROWS_PER_TILE, step=CHUNK)
    def _(c):
        g = pltpu.async_copy(w_hbm.at[idx_v.at[pl.ds(c, CHUNK)]], gat_v, gsem)
        xc = pltpu.async_copy(x_hbm.at[pl.ds(base + c, CHUNK)], x_v, xsem)
        g.wait()
        xc.wait()

        @pl.loop(0, CHUNK)
        def _(r):
            @pl.loop(0, D, step=LANES, unroll=8)
            def _(j):
                plsc.addupdate(gat_v.at[r, pl.ds(j, LANES)],
                               x_v[r, pl.ds(j, LANES)])

        pltpu.sync_copy(gat_v, o_hbm.at[pl.ds(base + c, CHUNK)])


@jax.jit
def _lookup_add(x2, idx, weight):
    kern = pl.kernel(
        _sc_body,
        out_type=jax.ShapeDtypeStruct((N, D), jnp.float32),
        mesh=plsc.VectorSubcoreMesh(core_axis_name="c", subcore_axis_name="s"),
        scratch_types=[
            pltpu.VMEM((ROWS_PER_TILE,), jnp.int32),
            pltpu.VMEM((CHUNK, D), jnp.float32),
            pltpu.VMEM((CHUNK, D), jnp.float32),
            pltpu.SemaphoreType.DMA,
            pltpu.SemaphoreType.DMA,
        ],
    )
    return kern(x2, idx, weight)


def kernel(x, index, weight):
    x2 = x.reshape(N, D)
    idx = index.reshape(N)
    out = _lookup_add(x2, idx, weight)
    return out.reshape(B, S, D)


# speedup vs baseline: 1.1818x; 1.1818x over previous
import functools
import jax
import jax.numpy as jnp
from jax import lax
from jax.experimental import pallas as pl
from jax.experimental.pallas import tpu as pltpu
from jax.experimental.pallas import tpu_sc as plsc

B, S, D = 4, 8192, 768
N = B * S
NC, NS, LANES = 2, 16, 16
NW = NC * NS
ROWS_PER_TILE = N // NW
CHUNK = 16
NCH = ROWS_PER_TILE // CHUNK
NBUF_IN = 4
NBUF_OUT = 2
PD = NBUF_IN - 1


def _sc_body(x_hbm, idx_hbm, w_hbm, o_hbm, idx_v,
             g0, g1, g2, g3, xb0, xb1, xb2, xb3, ob0, ob1,
             gsems, xsems, osems):
    gat = (g0, g1, g2, g3)
    xbs = (xb0, xb1, xb2, xb3)
    obs = (ob0, ob1)
    wid = lax.axis_index("s") * NC + lax.axis_index("c")
    base = wid * ROWS_PER_TILE
    pltpu.sync_copy(idx_hbm.at[pl.ds(base, ROWS_PER_TILE)], idx_v)

    def g_copy(c, b):
        return pltpu.make_async_copy(
            w_hbm.at[idx_v.at[pl.ds(c * CHUNK, CHUNK)]], gat[b], gsems.at[b])

    def x_copy(c, b):
        return pltpu.make_async_copy(
            x_hbm.at[pl.ds(base + c * CHUNK, CHUNK)], xbs[b], xsems.at[b])

    def o_copy(c, ob):
        return pltpu.make_async_copy(
            obs[ob], o_hbm.at[pl.ds(base + c * CHUNK, CHUNK)], osems.at[ob])

    for b in range(PD):
        g_copy(b, b).start()
        x_copy(b, b).start()

    @pl.loop(0, NCH, step=NBUF_IN)
    def _(c0):
        for b in range(NBUF_IN):
            c = c0 + b
            ob = b % NBUF_OUT

            @pl.when(c + PD < NCH)
            def _():
                g_copy(c + PD, (b + PD) % NBUF_IN).start()
                x_copy(c + PD, (b + PD) % NBUF_IN).start()

            g_copy(c, b).wait()
            x_copy(c, b).wait()

            @pl.when(c >= NBUF_OUT)
            def _():
                o_copy(c - NBUF_OUT, ob).wait()

            @pl.loop(0, CHUNK)
            def _(r):
                @pl.loop(0, D, step=LANES, unroll=8)
                def _(j):
                    obs[ob][r, pl.ds(j, LANES)] = (
                        gat[b][r, pl.ds(j, LANES)]
                        + xbs[b][r, pl.ds(j, LANES)])

            o_copy(c, ob).start()

    for t in range(NBUF_OUT):
        o_copy(NCH - NBUF_OUT + t, (NCH - NBUF_OUT + t) % NBUF_OUT).wait()


@jax.jit
def _lookup_add(x2, idx, weight):
    buf = pltpu.VMEM((CHUNK, D), jnp.float32)
    kern = pl.kernel(
        _sc_body,
        out_type=jax.ShapeDtypeStruct((N, D), jnp.float32),
        mesh=plsc.VectorSubcoreMesh(core_axis_name="c", subcore_axis_name="s"),
        scratch_types=[
            pltpu.VMEM((ROWS_PER_TILE,), jnp.int32),
            buf, buf, buf, buf,
            buf, buf, buf, buf,
            buf, buf,
            pltpu.SemaphoreType.DMA((NBUF_IN,)),
            pltpu.SemaphoreType.DMA((NBUF_IN,)),
            pltpu.SemaphoreType.DMA((NBUF_OUT,)),
        ],
    )
    return kern(x2, idx, weight)


def kernel(x, index, weight):
    x2 = x.reshape(N, D)
    idx = index.reshape(N)
    out = _lookup_add(x2, idx, weight)
    return out.reshape(B, S, D)


# speedup vs baseline: 1.4991x; 1.2686x over previous
import functools
import jax
import jax.numpy as jnp
from jax import lax
from jax.experimental import pallas as pl
from jax.experimental.pallas import tpu as pltpu
from jax.experimental.pallas import tpu_sc as plsc

B, S, D = 4, 8192, 768
N = B * S
NC, NS, LANES = 2, 16, 16
NW = NC * NS
ROWS_PER_TILE = N // NW
CHUNK = 16
NCH = ROWS_PER_TILE // CHUNK
NB = 4
PD = NB - 1
UNROLL = 16


def _sc_body(x_hbm, idx_hbm, w_hbm, o_hbm, idx_v,
             g0, g1, g2, g3, xb0, xb1, xb2, xb3,
             gsems, xsems, osems):
    gat = (g0, g1, g2, g3)
    xbs = (xb0, xb1, xb2, xb3)
    wid = lax.axis_index("s") * NC + lax.axis_index("c")
    base = wid * ROWS_PER_TILE
    pltpu.sync_copy(idx_hbm.at[pl.ds(base, ROWS_PER_TILE)], idx_v)

    def g_copy(c, s):
        return pltpu.make_async_copy(
            w_hbm.at[idx_v.at[pl.ds(c * CHUNK, CHUNK)]], gat[s], gsems.at[s])

    def x_copy(c, s):
        return pltpu.make_async_copy(
            x_hbm.at[pl.ds(base + c * CHUNK, CHUNK)], xbs[s], xsems.at[s])

    def o_copy(c, s):
        return pltpu.make_async_copy(
            xbs[s], o_hbm.at[pl.ds(base + c * CHUNK, CHUNK)], osems.at[s])

    for s in range(PD):
        g_copy(s, s).start()
        x_copy(s, s).start()

    @pl.loop(0, NCH, step=NB)
    def _(c0):
        for b in range(NB):
            c = c0 + b
            s_next = (b + PD) % NB

            @pl.when(c + PD < NCH)
            def _():
                g_copy(c + PD, s_next).start()

            g_copy(c, b).wait()
            x_copy(c, b).wait()

            @pl.loop(0, CHUNK)
            def _(r):
                @pl.loop(0, D, step=LANES, unroll=UNROLL)
                def _(j):
                    plsc.addupdate(xbs[b].at[r, pl.ds(j, LANES)],
                                   gat[b][r, pl.ds(j, LANES)])

            o_copy(c, b).start()

            @pl.when(c + PD < NCH)
            def _():
                @pl.when(c >= 1)
                def _():
                    o_copy(c - 1, s_next).wait()
                x_copy(c + PD, s_next).start()

    for t in range(PD + 1):
        cc = NCH - 1 - t
        o_copy(cc, cc % NB).wait()


@jax.jit
def _lookup_add(x2, idx, weight):
    buf = pltpu.VMEM((CHUNK, D), jnp.float32)
    kern = pl.kernel(
        _sc_body,
        out_type=jax.ShapeDtypeStruct((N, D), jnp.float32),
        mesh=plsc.VectorSubcoreMesh(core_axis_name="c", subcore_axis_name="s"),
        scratch_types=[
            pltpu.VMEM((ROWS_PER_TILE,), jnp.int32),
            buf, buf, buf, buf,
            buf, buf, buf, buf,
            pltpu.SemaphoreType.DMA((NB,)),
            pltpu.SemaphoreType.DMA((NB,)),
            pltpu.SemaphoreType.DMA((NB,)),
        ],
    )
    return kern(x2, idx, weight)


def kernel(x, index, weight):
    x2 = x.reshape(N, D)
    idx = index.reshape(N)
    out = _lookup_add(x2, idx, weight)
    return out.reshape(B, S, D)


# speedup vs baseline: 2.7207x; 1.8148x over previous
import functools
import jax
import jax.numpy as jnp
from jax import lax
from jax.experimental import pallas as pl
from jax.experimental.pallas import tpu as pltpu
from jax.experimental.pallas import tpu_sc as plsc

B, S, D = 4, 8192, 768
N = B * S
NC, NS, LANES = 2, 16, 16
NW = NC * NS
ROWS_PER_TILE = N // NW
CHUNK = 16
NCH = ROWS_PER_TILE // CHUNK
NB = 4
PD = NB - 1
KB = 8


def _sc_body(x_hbm, idx_hbm, w_hbm, o_hbm, idx_v,
             g0, g1, g2, g3, xb0, xb1, xb2, xb3,
             gsems, xsems, osems):
    gat = (g0, g1, g2, g3)
    xbs = (xb0, xb1, xb2, xb3)
    wid = lax.axis_index("s") * NC + lax.axis_index("c")
    base = wid * ROWS_PER_TILE
    pltpu.sync_copy(idx_hbm.at[pl.ds(base, ROWS_PER_TILE)], idx_v)

    def g_copy(c, s):
        return pltpu.make_async_copy(
            w_hbm.at[idx_v.at[pl.ds(c * CHUNK, CHUNK)]], gat[s], gsems.at[s])

    def x_copy(c, s):
        return pltpu.make_async_copy(
            x_hbm.at[pl.ds(base + c * CHUNK, CHUNK)], xbs[s], xsems.at[s])

    def o_copy(c, s):
        return pltpu.make_async_copy(
            xbs[s], o_hbm.at[pl.ds(base + c * CHUNK, CHUNK)], osems.at[s])

    for s in range(PD):
        g_copy(s, s).start()
        x_copy(s, s).start()

    @pl.loop(0, NCH, step=NB)
    def _(c0):
        for b in range(NB):
            c = c0 + b
            s_next = (b + PD) % NB

            @pl.when(c + PD < NCH)
            def _():
                g_copy(c + PD, s_next).start()

            g_copy(c, b).wait()
            x_copy(c, b).wait()

            @pl.loop(0, CHUNK)
            def _(r):
                @pl.loop(0, D, step=LANES * KB)
                def _(j):
                    vals = [gat[b][r, pl.ds(j + k * LANES, LANES)]
                            for k in range(KB)]
                    for k in range(KB):
                        plsc.addupdate(xbs[b].at[r, pl.ds(j + k * LANES,
                                                          LANES)], vals[k])

            o_copy(c, b).start()

            @pl.when(c + PD < NCH)
            def _():
                @pl.when(c >= 1)
                def _():
                    o_copy(c - 1, s_next).wait()
                x_copy(c + PD, s_next).start()

    for t in range(PD + 1):
        cc = NCH - 1 - t
        o_copy(cc, cc % NB).wait()


@jax.jit
def _lookup_add(x2, idx, weight):
    buf = pltpu.VMEM((CHUNK, D), jnp.float32)
    kern = pl.kernel(
        _sc_body,
        out_type=jax.ShapeDtypeStruct((N, D), jnp.float32),
        mesh=plsc.VectorSubcoreMesh(core_axis_name="c", subcore_axis_name="s"),
        scratch_types=[
            pltpu.VMEM((ROWS_PER_TILE,), jnp.int32),
            buf, buf, buf, buf,
            buf, buf, buf, buf,
            pltpu.SemaphoreType.DMA((NB,)),
            pltpu.SemaphoreType.DMA((NB,)),
            pltpu.SemaphoreType.DMA((NB,)),
        ],
    )
    return kern(x2, idx, weight)


def kernel(x, index, weight):
    x2 = x.reshape(N, D)
    idx = index.reshape(N)
    out = _lookup_add(x2, idx, weight)
    return out.reshape(B, S, D)
